# Initial kernel scaffold; baseline (speedup 1.0000x reference)
import jax
import jax.numpy as jnp
from jax.experimental import pallas as pl


def kernel(input_ids, encoder_embed_scale, decoder_input_ids, decoder_embed_scale, table):
    raise NotImplementedError("write your pallas kernel here")


# speedup vs baseline: 1.1021x; 1.1021x over previous
import functools

import jax
import jax.numpy as jnp
from jax import lax
from jax.experimental import pallas as pl
from jax.experimental.pallas import tpu as pltpu
from jax.experimental.pallas import tpu_sc as plsc

NUM_EMBEDDINGS = 100000
EMBED_DIM = 128
TOKENS = 4 * (2048 + 2048)
NC, NS, L = 2, 16, 16
NW = NC * NS
B_PER_W = TOKENS // NW
CHUNK = 128
N_CHUNKS = B_PER_W // CHUNK


@functools.partial(
    pl.kernel,
    out_type=jax.ShapeDtypeStruct((TOKENS, EMBED_DIM), jnp.float32),
    mesh=plsc.VectorSubcoreMesh(core_axis_name="c", subcore_axis_name="s"),
    scratch_types=[
        pltpu.VMEM((N_CHUNKS, CHUNK), jnp.int32),
        pltpu.VMEM((B_PER_W, EMBED_DIM), jnp.float32),
        pltpu.VMEM((L,), jnp.float32),
        pltpu.SemaphoreType.DMA,
    ],
)
def _embed_kernel(ids_hbm, scales_hbm, table_hbm, out_hbm, idx_v, rows_v, scale_v, sem):
    wid = lax.axis_index("s") * NC + lax.axis_index("c")
    base_row = wid * N_CHUNKS

    pltpu.sync_copy(ids_hbm.at[pl.ds(base_row, N_CHUNKS)], idx_v)
    pltpu.sync_copy(scales_hbm.at[wid], scale_v)

    copies = []
    for j in range(N_CHUNKS):
        copies.append(
            pltpu.async_copy(
                table_hbm.at[idx_v.at[j]],
                rows_v.at[pl.ds(j * CHUNK, CHUNK)],
                sem,
            )
        )
    for c in copies:
        c.wait()

    s = scale_v[...]

    def scale_row(i, carry):
        for cslice in range(EMBED_DIM // L):
            sl = pl.ds(cslice * L, L)
            rows_v[i, sl] = rows_v[i, sl] * s
        return carry

    lax.fori_loop(0, B_PER_W, scale_row, 0)

    pltpu.sync_copy(rows_v, out_hbm.at[pl.ds(wid * B_PER_W, B_PER_W)])


def kernel(input_ids, encoder_embed_scale, decoder_input_ids, decoder_embed_scale, table):
    batch, enc_len = input_ids.shape
    dec_len = decoder_input_ids.shape[1]
    ids = jnp.concatenate(
        [input_ids.reshape(-1), decoder_input_ids.reshape(-1)]
    ).astype(jnp.int32)
    ids2d = ids.reshape(TOKENS // CHUNK, CHUNK)

    enc_tokens = batch * enc_len
    is_enc = (jnp.arange(NW) * B_PER_W) < enc_tokens
    scales = jnp.where(
        is_enc,
        encoder_embed_scale.astype(jnp.float32),
        decoder_embed_scale.astype(jnp.float32),
    )
    scales = jnp.broadcast_to(scales[:, None], (NW, L)).astype(jnp.float32)

    out = _embed_kernel(ids2d, scales, table)
    enc = out[:enc_tokens].reshape(batch, enc_len, EMBED_DIM)
    dec = out[enc_tokens:].reshape(batch, dec_len, EMBED_DIM)
    return (enc, dec)


# speedup vs baseline: 1.2858x; 1.1667x over previous
import functools

import jax
import jax.numpy as jnp
from jax import lax
from jax.experimental import pallas as pl
from jax.experimental.pallas import tpu as pltpu
from jax.experimental.pallas import tpu_sc as plsc

EMBED_DIM = 128
SIDE_TOKENS = 4 * 2048
NC, NS, L = 2, 16, 16
NW = NC * NS
CHUNK = 128
CHUNKS_PER_SIDE = SIDE_TOKENS // (NW * CHUNK)
N_CHUNKS = 2 * CHUNKS_PER_SIDE
ROWS_UNROLL = 4


@functools.partial(
    pl.kernel,
    out_type=(
        jax.ShapeDtypeStruct((SIDE_TOKENS, EMBED_DIM), jnp.float32),
        jax.ShapeDtypeStruct((SIDE_TOKENS, EMBED_DIM), jnp.float32),
    ),
    mesh=plsc.VectorSubcoreMesh(core_axis_name="c", subcore_axis_name="s"),
    scratch_types=[
        pltpu.VMEM((N_CHUNKS, CHUNK), jnp.int32),
        pltpu.VMEM((N_CHUNKS * CHUNK, EMBED_DIM), jnp.float32),
        pltpu.VMEM((L,), jnp.float32),
        pltpu.VMEM((L,), jnp.float32),
        pltpu.SemaphoreType.DMA,
        pltpu.SemaphoreType.DMA,
        pltpu.SemaphoreType.DMA,
        pltpu.SemaphoreType.DMA,
        pltpu.SemaphoreType.DMA,
    ],
)
def _embed_kernel(enc_ids, dec_ids, enc_s, dec_s, table,
                  enc_out, dec_out,
                  idx_v, rows_v, senc_v, sdec_v, g0, g1, g2, g3, osem):
    wid = lax.axis_index("s") * NC + lax.axis_index("c")

    pltpu.sync_copy(enc_ids.at[pl.ds(wid * CHUNKS_PER_SIDE, CHUNKS_PER_SIDE)],
                    idx_v.at[pl.ds(0, CHUNKS_PER_SIDE)])
    pltpu.sync_copy(dec_ids.at[pl.ds(wid * CHUNKS_PER_SIDE, CHUNKS_PER_SIDE)],
                    idx_v.at[pl.ds(CHUNKS_PER_SIDE, CHUNKS_PER_SIDE)])
    pltpu.sync_copy(enc_s, senc_v)
    pltpu.sync_copy(dec_s, sdec_v)

    gsems = [g0, g1, g2, g3]
    gathers = [
        pltpu.async_copy(table.at[idx_v.at[j]],
                         rows_v.at[pl.ds(j * CHUNK, CHUNK)], gsems[j])
        for j in range(N_CHUNKS)
    ]

    writes = []
    for j in range(N_CHUNKS):
        gathers[j].wait()
        is_enc = j < CHUNKS_PER_SIDE
        s = (senc_v if is_enc else sdec_v)[...]
        base = j * CHUNK

        def scale_rows(i, carry, base=base, s=s):
            row = base + i * ROWS_UNROLL
            for r in range(ROWS_UNROLL):
                for cs in range(EMBED_DIM // L):
                    sl = pl.ds(cs * L, L)
                    rows_v[row + r, sl] = rows_v[row + r, sl] * s
            return carry

        lax.fori_loop(0, CHUNK // ROWS_UNROLL, scale_rows, 0)

        dst = enc_out if is_enc else dec_out
        off = (wid * CHUNKS_PER_SIDE + (j % CHUNKS_PER_SIDE)) * CHUNK
        writes.append(
            pltpu.async_copy(rows_v.at[pl.ds(base, CHUNK)],
                             dst.at[pl.ds(off, CHUNK)], osem)
        )
    for w in writes:
        w.wait()


def kernel(input_ids, encoder_embed_scale, decoder_input_ids, decoder_embed_scale, table):
    batch, enc_len = input_ids.shape
    dec_len = decoder_input_ids.shape[1]
    enc_ids = input_ids.reshape(SIDE_TOKENS // CHUNK, CHUNK).astype(jnp.int32)
    dec_ids = decoder_input_ids.reshape(SIDE_TOKENS // CHUNK, CHUNK).astype(jnp.int32)
    enc_s = jnp.full((L,), encoder_embed_scale, jnp.float32)
    dec_s = jnp.full((L,), decoder_embed_scale, jnp.float32)

    enc, dec = _embed_kernel(enc_ids, dec_ids, enc_s, dec_s, table)
    return (enc.reshape(batch, enc_len, EMBED_DIM),
            dec.reshape(batch, dec_len, EMBED_DIM))


# speedup vs baseline: 1.4550x; 1.1316x over previous
import functools

import jax
import jax.numpy as jnp
from jax import lax
from jax.experimental import pallas as pl
from jax.experimental.pallas import tpu as pltpu
from jax.experimental.pallas import tpu_sc as plsc

EMBED_DIM = 128
BATCH = 4
SEQ = 2048
SIDE_TOKENS = BATCH * SEQ
NC, NS, L = 2, 16, 16
NW = NC * NS
CHUNK = 128
PER_SIDE = SIDE_TOKENS // NW
CHUNKS_PER_SIDE = PER_SIDE // CHUNK
N_CHUNKS = 2 * CHUNKS_PER_SIDE
W_PER_ROW = SEQ // PER_SIDE
ROWS_UNROLL = 2


@functools.partial(
    pl.kernel,
    out_type=(
        jax.ShapeDtypeStruct((SIDE_TOKENS, EMBED_DIM), jnp.float32),
        jax.ShapeDtypeStruct((SIDE_TOKENS, EMBED_DIM), jnp.float32),
    ),
    mesh=plsc.VectorSubcoreMesh(core_axis_name="c", subcore_axis_name="s"),
    scratch_types=[
        pltpu.VMEM((N_CHUNKS * CHUNK,), jnp.int32),
        pltpu.VMEM((N_CHUNKS * CHUNK, EMBED_DIM), jnp.float32),
        pltpu.VMEM((2, L), jnp.float32),
        pltpu.SemaphoreType.DMA,
        pltpu.SemaphoreType.DMA,
        pltpu.SemaphoreType.DMA,
        pltpu.SemaphoreType.DMA,
        pltpu.SemaphoreType.DMA,
    ],
)
def _embed_kernel(enc_ids, dec_ids, scales, table,
                  enc_out, dec_out,
                  idx_v, rows_v, scale_v, g0, g1, g2, g3, osem):
    wid = lax.axis_index("s") * NC + lax.axis_index("c")
    b = wid // W_PER_ROW
    col = (wid % W_PER_ROW) * PER_SIDE

    gsems = [g0, g1, g2, g3]
    gathers = []
    for side, ids in enumerate((enc_ids, dec_ids)):
        off = side * PER_SIDE
        pltpu.sync_copy(ids.at[b, pl.ds(col, PER_SIDE)],
                        idx_v.at[pl.ds(off, PER_SIDE)])
        for jj in range(CHUNKS_PER_SIDE):
            j = side * CHUNKS_PER_SIDE + jj
            gathers.append(
                pltpu.async_copy(table.at[idx_v.at[pl.ds(j * CHUNK, CHUNK)]],
                                 rows_v.at[pl.ds(j * CHUNK, CHUNK)], gsems[j])
            )
    pltpu.sync_copy(scales, scale_v)

    writes = []
    for j in range(N_CHUNKS):
        gathers[j].wait()
        side = j // CHUNKS_PER_SIDE
        s = scale_v[side, :]
        base = j * CHUNK

        def scale_rows(i, carry, base=base, s=s):
            row = base + i * ROWS_UNROLL
            for r in range(ROWS_UNROLL):
                for cs in range(EMBED_DIM // L):
                    sl = pl.ds(cs * L, L)
                    rows_v[row + r, sl] = rows_v[row + r, sl] * s
            return carry

        lax.fori_loop(0, CHUNK // ROWS_UNROLL, scale_rows, 0)

        dst = enc_out if side == 0 else dec_out
        off = wid * PER_SIDE + (j % CHUNKS_PER_SIDE) * CHUNK
        writes.append(
            pltpu.async_copy(rows_v.at[pl.ds(base, CHUNK)],
                             dst.at[pl.ds(off, CHUNK)], osem)
        )
    for w in writes:
        w.wait()


def kernel(input_ids, encoder_embed_scale, decoder_input_ids, decoder_embed_scale, table):
    batch, enc_len = input_ids.shape
    dec_len = decoder_input_ids.shape[1]
    scales = jnp.broadcast_to(
        jnp.stack([encoder_embed_scale, decoder_embed_scale]).astype(jnp.float32)[:, None],
        (2, L),
    )
    enc, dec = _embed_kernel(input_ids.astype(jnp.int32),
                             decoder_input_ids.astype(jnp.int32),
                             scales, table)
    return (enc.reshape(batch, enc_len, EMBED_DIM),
            dec.reshape(batch, dec_len, EMBED_DIM))
